# baseline (speedup 1.0000x reference)
import jax
import jax.numpy as jnp
from jax.experimental import pallas as pl
from jax.experimental.pallas import tpu as pltpu

N = 320000
D = 128
S = 1024
BN = 512
NB = N // BN


def _fused_kernel(bounds_ref, x_ref, pos_ref, ids_ref, wx_ref, wp_ref,
                  out_ref):
    i = pl.program_id(0)

    @pl.when(i == 0)
    def _init():
        out_ref[...] = jnp.zeros_like(out_ref)

    h = jnp.dot(x_ref[...], wx_ref[...], preferred_element_type=jnp.float32)
    h += jnp.dot(pos_ref[...], wp_ref[...], preferred_element_type=jnp.float32)
    h = jnp.maximum(h, 0.0)

    ids = ids_ref[0]
    s_lo = bounds_ref[i, 0]
    s_hi = bounds_ref[i, 1]

    def body(s, _):
        col = jnp.max(jnp.where(ids == s, h, 0.0), axis=0, keepdims=True)
        cur = out_ref[pl.ds(s, 1), :]
        out_ref[pl.ds(s, 1), :] = jnp.maximum(cur, col)
        return 0

    jax.lax.fori_loop(s_lo, s_hi + 1, body, 0, unroll=False)


def kernel(x, pos, batch, W1, b1):
    ids = batch.astype(jnp.int32)
    bounds = jnp.stack([ids[::BN], ids[BN - 1::BN]], axis=1)
    ids3 = ids.reshape(NB, BN, 1)
    wx = W1[:D]
    pos4 = jnp.concatenate([pos, jnp.ones((N, 1), dtype=pos.dtype)], axis=1)
    wp4 = jnp.concatenate([W1[D:], b1.reshape(1, 128)], axis=0)

    grid_spec = pltpu.PrefetchScalarGridSpec(
        num_scalar_prefetch=1,
        grid=(NB,),
        in_specs=[
            pl.BlockSpec((BN, D), lambda i, b_: (i, 0)),
            pl.BlockSpec((BN, 4), lambda i, b_: (i, 0)),
            pl.BlockSpec((1, BN, 1), lambda i, b_: (i, 0, 0)),
            pl.BlockSpec((D, 128), lambda i, b_: (0, 0)),
            pl.BlockSpec((4, 128), lambda i, b_: (0, 0)),
        ],
        out_specs=pl.BlockSpec((S, 128), lambda i, b_: (0, 0)),
    )

    pooled = pl.pallas_call(
        _fused_kernel,
        grid_spec=grid_spec,
        out_shape=jax.ShapeDtypeStruct((S, 128), jnp.float32),
    )(bounds, x, pos4, ids3, wx, wp4)

    pos_out = jnp.zeros((S, 3), dtype=pos.dtype)
    batch_out = jnp.arange(S, dtype=batch.dtype)
    return pooled, pos_out, batch_out


# speedup vs baseline: 1.0811x; 1.0811x over previous
import jax
import jax.numpy as jnp
from jax.experimental import pallas as pl
from jax.experimental.pallas import tpu as pltpu

N = 320000
D = 128
S = 1024
BN = 512
NB = N // BN
W = 4


def _fused_kernel(bounds_ref, x_ref, pos_ref, ids_ref, wx_ref, wp_ref,
                  out_ref):
    i = pl.program_id(0)

    @pl.when(i == 0)
    def _init():
        out_ref[...] = jnp.zeros_like(out_ref)

    h = jnp.dot(x_ref[...], wx_ref[...], preferred_element_type=jnp.float32)
    h += jnp.dot(pos_ref[...], wp_ref[...], preferred_element_type=jnp.float32)

    ids = ids_ref[0]
    s_lo = bounds_ref[i, 0]
    s_hi = bounds_ref[i, 1]

    base = jnp.minimum(s_lo, S - W)
    win = jnp.concatenate(
        [jnp.max(jnp.where(ids == base + k, h, 0.0), axis=0, keepdims=True)
         for k in range(W)], axis=0)
    cur = out_ref[pl.ds(base, W), :]
    out_ref[pl.ds(base, W), :] = jnp.maximum(cur, win)

    @pl.when(s_hi >= base + W)
    def _resid():
        def body(s, _):
            col = jnp.max(jnp.where(ids == s, h, 0.0), axis=0, keepdims=True)
            cur = out_ref[pl.ds(s, 1), :]
            out_ref[pl.ds(s, 1), :] = jnp.maximum(cur, col)
            return 0

        jax.lax.fori_loop(base + W, s_hi + 1, body, 0, unroll=False)


def kernel(x, pos, batch, W1, b1):
    ids = batch.astype(jnp.int32)
    bounds = jnp.stack([ids[::BN], ids[BN - 1::BN]], axis=1)
    ids3 = ids.reshape(NB, BN, 1)
    wx = W1[:D]
    pos4 = jnp.concatenate([pos, jnp.ones((N, 1), dtype=pos.dtype)], axis=1)
    wp4 = jnp.concatenate([W1[D:], b1.reshape(1, 128)], axis=0)

    grid_spec = pltpu.PrefetchScalarGridSpec(
        num_scalar_prefetch=1,
        grid=(NB,),
        in_specs=[
            pl.BlockSpec((BN, D), lambda i, b_: (i, 0)),
            pl.BlockSpec((BN, 4), lambda i, b_: (i, 0)),
            pl.BlockSpec((1, BN, 1), lambda i, b_: (i, 0, 0)),
            pl.BlockSpec((D, 128), lambda i, b_: (0, 0)),
            pl.BlockSpec((4, 128), lambda i, b_: (0, 0)),
        ],
        out_specs=pl.BlockSpec((S, 128), lambda i, b_: (0, 0)),
    )

    pooled = pl.pallas_call(
        _fused_kernel,
        grid_spec=grid_spec,
        out_shape=jax.ShapeDtypeStruct((S, 128), jnp.float32),
    )(bounds, x, pos4, ids3, wx, wp4)

    pos_out = jnp.zeros((S, 3), dtype=pos.dtype)
    batch_out = jnp.arange(S, dtype=batch.dtype)
    return pooled, pos_out, batch_out
